# baseline (speedup 1.0000x reference)
import jax
import jax.numpy as jnp
from jax import lax
from jax.experimental import pallas as pl
from jax.experimental.pallas import tpu as pltpu
from jax.experimental.pallas import tpu_sc as plsc

NC, NS = 2, 16
NW = NC * NS
BB = 128
DIM = 64


def _sc_body(idx4_hbm, tok_hbm, pos_hbm, out_hbm,
             idx_v, rows0, rows1, tb0, tb1, pos_v,
             gsem0, gsem1, osem0, osem1):
    seq = idx4_hbm.shape[0] * idx4_hbm.shape[2]
    wid = lax.axis_index("s") * NC + lax.axis_index("c")

    rows = (rows0, rows1)
    tbs = (tb0, tb1)
    gsems = (gsem0, gsem1)
    osems = (osem0, osem1)

    pltpu.sync_copy(pos_hbm, pos_v)
    pltpu.sync_copy(idx4_hbm.at[:, pl.ds(wid, 1)], idx_v)

    iotas = [lax.iota(jnp.int32, 16) + 16 * j for j in range(BB // 16)]

    def fire_gather(s, b):
        pltpu.async_copy(tok_hbm.at[idx_v.at[s // 8, 0, s % 8]], rows[b], gsems[b])

    def drain_gather(b):
        pltpu.make_async_copy(tok_hbm.at[pl.ds(0, BB)], rows[b], gsems[b]).wait()

    def fire_out(s, b):
        pltpu.async_copy(tbs[b], out_hbm.at[s, :, pl.ds(wid, 1)], osems[b])

    def drain_out(b):
        pltpu.make_async_copy(tbs[b], out_hbm.at[0, :, pl.ds(0, 1)], osems[b]).wait()

    def compute(s, b):
        rb, tb = rows[b], tbs[b]
        s_splat = jnp.full((16,), s, jnp.int32)

        @plsc.parallel_loop(0, DIM // 8, 1)
        def _(k):
            for r in range(8):
                d = 8 * k + r
                d_splat = jnp.full((16,), d, jnp.int32)
                p = plsc.load_gather(pos_v, [s_splat, d_splat])
                for j in range(BB // 16):
                    g = plsc.load_gather(rb, [iotas[j], d_splat])
                    tb[k, 0, r, pl.ds(16 * j, 16)] = g * 8.0 + p

    fire_gather(0, 0)
    fire_gather(1, 1)
    drain_gather(0)
    compute(0, 0)
    fire_out(0, 0)
    fire_gather(2, 0)
    drain_gather(1)
    compute(1, 1)
    fire_out(1, 1)
    fire_gather(3, 1)

    def pair(t, _):
        s = 2 * t + 2
        drain_out(0)
        drain_gather(0)
        compute(s, 0)
        fire_out(s, 0)
        fire_gather(s + 2, 0)

        drain_out(1)
        drain_gather(1)
        compute(s + 1, 1)
        fire_out(s + 1, 1)
        fire_gather(s + 3, 1)
        return _

    lax.fori_loop(0, (seq - 4) // 2, pair, 0)

    for s, b in ((seq - 2, 0), (seq - 1, 1)):
        drain_out(b)
        drain_gather(b)
        compute(s, b)
        fire_out(s, b)
    drain_out(0)
    drain_out(1)


def kernel(inputs, token_table, position_table):
    batch, seq = inputs.shape
    vocab, dim = token_table.shape
    idx4 = (
        inputs.astype(jnp.int32)
        .reshape(batch // BB, BB, seq // 8, 8)
        .transpose(2, 0, 3, 1)
    )

    mesh = plsc.VectorSubcoreMesh(
        core_axis_name="c", subcore_axis_name="s", num_cores=NC, num_subcores=NS
    )
    call = pl.kernel(
        _sc_body,
        out_type=jax.ShapeDtypeStruct((seq, dim // 8, batch // BB, 8, BB), jnp.float32),
        name="emb_kernel",
        mesh=mesh,
        scratch_types=[
            pltpu.VMEM((seq // 8, 1, 8, BB), jnp.int32),
            pltpu.VMEM((BB, dim), jnp.float32),
            pltpu.VMEM((BB, dim), jnp.float32),
            pltpu.VMEM((dim // 8, 1, 8, BB), jnp.float32),
            pltpu.VMEM((dim // 8, 1, 8, BB), jnp.float32),
            pltpu.VMEM((seq, dim), jnp.float32),
            pltpu.SemaphoreType.DMA,
            pltpu.SemaphoreType.DMA,
            pltpu.SemaphoreType.DMA,
            pltpu.SemaphoreType.DMA,
        ],
        compiler_params=pltpu.CompilerParams(
            use_tc_tiling_on_sc=False, needs_layout_passes=False
        ),
    )
    a = call(idx4, token_table, position_table)
    return a.transpose(2, 4, 0, 1, 3).reshape(batch, seq, dim)


# speedup vs baseline: 1.2264x; 1.2264x over previous
import jax
import jax.numpy as jnp
from jax import lax
from jax.experimental import pallas as pl
from jax.experimental.pallas import tpu as pltpu
from jax.experimental.pallas import tpu_sc as plsc

NC, NS = 2, 16
NW = NC * NS
BB = 128
DIM = 64


def _sc_body(idx4_hbm, tok_hbm, pos_hbm, out_hbm,
             idx_v, rows0, rows1, tb0, tb1, pos_v,
             gsem0, gsem1, osem0, osem1):
    seq = idx4_hbm.shape[0] * idx4_hbm.shape[2]
    wid = lax.axis_index("s") * NC + lax.axis_index("c")

    rows = (rows0, rows1)
    tbs = (tb0, tb1)
    gsems = (gsem0, gsem1)
    osems = (osem0, osem1)

    pltpu.sync_copy(pos_hbm, pos_v)
    pltpu.sync_copy(idx4_hbm.at[:, pl.ds(wid, 1)], idx_v)

    def fire_gather(s, b):
        pltpu.async_copy(tok_hbm.at[idx_v.at[s // 8, 0, s % 8]], rows[b], gsems[b])

    def drain_gather(b):
        pltpu.make_async_copy(tok_hbm.at[pl.ds(0, BB)], rows[b], gsems[b]).wait()

    def fire_out(s, b):
        pltpu.async_copy(tbs[b], out_hbm.at[s, :, pl.ds(wid, 1)], osems[b])

    def drain_out(b):
        pltpu.make_async_copy(tbs[b], out_hbm.at[0, :, pl.ds(0, 1)], osems[b]).wait()

    def compute(s, b):
        rb, tb = rows[b], tbs[b]
        iota = lax.iota(jnp.int32, 16)
        zero16 = jnp.zeros((16,), jnp.int32)
        s_splat = jnp.full((16,), s, jnp.int32)

        def tbody(t, _):
            m = (t + iota) % 16
            w_base = (m % 8) * 128 + iota
            k_base = m // 8
            for u in range(DIM // 16):
                dvec = 16 * u + m
                pg = plsc.load_gather(pos_v, [s_splat, dvec])
                kvec = 2 * u + k_base
                for j in range(BB // 16):
                    cvec = 16 * j + iota
                    g = plsc.load_gather(rb, [cvec, dvec])
                    plsc.store_scatter(tb, [kvec, zero16, 16 * j + w_base],
                                       g * 8.0 + pg)
            return _

        lax.fori_loop(0, 16, tbody, 0)

    fire_gather(0, 0)
    fire_gather(1, 1)

    def step(s, b):
        @pl.when(s >= 2)
        def _():
            drain_out(b)

        drain_gather(b)
        compute(s, b)
        fire_out(s, b)

        @pl.when(s + 2 < seq)
        def _():
            fire_gather(s + 2, b)

    def pair(t, _):
        step(2 * t, 0)
        step(2 * t + 1, 1)
        return _

    lax.fori_loop(0, seq // 2, pair, 0)
    drain_out(0)
    drain_out(1)


def kernel(inputs, token_table, position_table):
    batch, seq = inputs.shape
    vocab, dim = token_table.shape
    idx4 = (
        inputs.astype(jnp.int32)
        .reshape(batch // BB, BB, seq // 8, 8)
        .transpose(2, 0, 3, 1)
    )

    mesh = plsc.VectorSubcoreMesh(
        core_axis_name="c", subcore_axis_name="s", num_cores=NC, num_subcores=NS
    )
    call = pl.kernel(
        _sc_body,
        out_type=jax.ShapeDtypeStruct((seq, dim // 8, batch // BB, 8 * BB), jnp.float32),
        name="emb_kernel",
        mesh=mesh,
        scratch_types=[
            pltpu.VMEM((seq // 8, 1, 8, BB), jnp.int32),
            pltpu.VMEM((BB, dim), jnp.float32),
            pltpu.VMEM((BB, dim), jnp.float32),
            pltpu.VMEM((dim // 8, 1, 8 * BB), jnp.float32),
            pltpu.VMEM((dim // 8, 1, 8 * BB), jnp.float32),
            pltpu.VMEM((seq, dim), jnp.float32),
            pltpu.SemaphoreType.DMA,
            pltpu.SemaphoreType.DMA,
            pltpu.SemaphoreType.DMA,
            pltpu.SemaphoreType.DMA,
        ],
        compiler_params=pltpu.CompilerParams(
            use_tc_tiling_on_sc=False, needs_layout_passes=False
        ),
    )
    a = call(idx4, token_table, position_table)
    return (
        a.reshape(seq, dim // 8, batch // BB, 8, BB)
        .transpose(2, 4, 0, 1, 3)
        .reshape(batch, seq, dim)
    )


# speedup vs baseline: 1.3352x; 1.0887x over previous
import jax
import jax.numpy as jnp
from jax import lax
from jax.experimental import pallas as pl
from jax.experimental.pallas import tpu as pltpu
from jax.experimental.pallas import tpu_sc as plsc

NC, NS = 2, 16
NW = NC * NS
BB = 128
DIM = 64


def _sc_body(idx4_hbm, tok_hbm, pos_hbm, out_hbm,
             idx_v, rows0, rows1, tb0, tb1, pos_v,
             gsem0, gsem1, osem0, osem1):
    seq = idx4_hbm.shape[0] * idx4_hbm.shape[2]
    wid = lax.axis_index("s") * NC + lax.axis_index("c")

    rows = (rows0, rows1)
    tbs = (tb0, tb1)
    gsems = (gsem0, gsem1)
    osems = (osem0, osem1)

    pltpu.sync_copy(pos_hbm, pos_v)
    pltpu.sync_copy(idx4_hbm.at[:, pl.ds(wid, 1)], idx_v)

    def fire_gather(s, b):
        pltpu.async_copy(tok_hbm.at[idx_v.at[s // 8, 0, s % 8]], rows[b], gsems[b])

    def drain_gather(b):
        pltpu.make_async_copy(tok_hbm.at[pl.ds(0, BB)], rows[b], gsems[b]).wait()

    def fire_out(s, b):
        pltpu.async_copy(tbs[b], out_hbm.at[s, :, pl.ds(wid, 1)], osems[b])

    def drain_out(b):
        pltpu.make_async_copy(tbs[b], out_hbm.at[0, :, pl.ds(0, 1)], osems[b]).wait()

    def compute(s, b):
        rb, tb = rows[b], tbs[b]
        iota = lax.iota(jnp.int32, 16)
        zero16 = jnp.zeros((16,), jnp.int32)
        s_splat = jnp.full((16,), s, jnp.int32)

        @plsc.parallel_loop(0, 16, 1)
        def _(t):
            m = (t + iota) % 16
            w_base = (m % 8) * 128 + iota
            k_base = m // 8
            for u in range(DIM // 16):
                dvec = 16 * u + m
                pg = plsc.load_gather(pos_v, [s_splat, dvec])
                kvec = 2 * u + k_base
                for j in range(BB // 16):
                    cvec = 16 * j + iota
                    g = plsc.load_gather(rb, [cvec, dvec])
                    plsc.store_scatter(tb, [kvec, zero16, 16 * j + w_base],
                                       g * 8.0 + pg)

    fire_gather(0, 0)
    fire_gather(1, 1)

    def step(s, b):
        @pl.when(s >= 2)
        def _():
            drain_out(b)

        drain_gather(b)
        compute(s, b)
        fire_out(s, b)

        @pl.when(s + 2 < seq)
        def _():
            fire_gather(s + 2, b)

    def pair(t, _):
        step(2 * t, 0)
        step(2 * t + 1, 1)
        return _

    lax.fori_loop(0, seq // 2, pair, 0)
    drain_out(0)
    drain_out(1)


def kernel(inputs, token_table, position_table):
    batch, seq = inputs.shape
    vocab, dim = token_table.shape
    idx4 = (
        inputs.astype(jnp.int32)
        .reshape(batch // BB, BB, seq // 8, 8)
        .transpose(2, 0, 3, 1)
    )

    mesh = plsc.VectorSubcoreMesh(
        core_axis_name="c", subcore_axis_name="s", num_cores=NC, num_subcores=NS
    )
    call = pl.kernel(
        _sc_body,
        out_type=jax.ShapeDtypeStruct((seq, dim // 8, batch // BB, 8 * BB), jnp.float32),
        name="emb_kernel",
        mesh=mesh,
        scratch_types=[
            pltpu.VMEM((seq // 8, 1, 8, BB), jnp.int32),
            pltpu.VMEM((BB, dim), jnp.float32),
            pltpu.VMEM((BB, dim), jnp.float32),
            pltpu.VMEM((dim // 8, 1, 8 * BB), jnp.float32),
            pltpu.VMEM((dim // 8, 1, 8 * BB), jnp.float32),
            pltpu.VMEM((seq, dim), jnp.float32),
            pltpu.SemaphoreType.DMA,
            pltpu.SemaphoreType.DMA,
            pltpu.SemaphoreType.DMA,
            pltpu.SemaphoreType.DMA,
        ],
        compiler_params=pltpu.CompilerParams(
            use_tc_tiling_on_sc=False, needs_layout_passes=False
        ),
    )
    a = call(idx4, token_table, position_table)
    return (
        a.reshape(seq, dim // 8, batch // BB, 8, BB)
        .transpose(2, 4, 0, 1, 3)
        .reshape(batch, seq, dim)
    )


# speedup vs baseline: 1.3628x; 1.0207x over previous
import jax
import jax.numpy as jnp
from jax import lax
from jax.experimental import pallas as pl
from jax.experimental.pallas import tpu as pltpu
from jax.experimental.pallas import tpu_sc as plsc

NC, NS = 2, 16
NW = NC * NS
BB = 128
DIM = 64


def _sc_body(idx4_hbm, tok_hbm, pos_hbm, out_hbm,
             idx_v, rows0, rows1, tb0, tb1, pos_v, skew_v,
             gsem0, gsem1, osem0, osem1):
    seq = idx4_hbm.shape[0] * idx4_hbm.shape[2]
    wid = lax.axis_index("s") * NC + lax.axis_index("c")

    rows = (rows0, rows1)
    tbs = (tb0, tb1)
    gsems = (gsem0, gsem1)
    osems = (osem0, osem1)

    pltpu.sync_copy(pos_hbm, pos_v)
    pltpu.sync_copy(idx4_hbm.at[:, pl.ds(wid, 1)], idx_v)

    iota = lax.iota(jnp.int32, 16)
    zero16 = jnp.zeros((16,), jnp.int32)

    @plsc.parallel_loop(0, 16, 1)
    def _(t):
        m = (t + iota) % 16
        skew_v[0, t, pl.ds(0, 16)] = m
        skew_v[1, t, pl.ds(0, 16)] = m // 8
        skew_v[2, t, pl.ds(0, 16)] = (m % 8) * 128 + iota

    def fire_gather(s, b):
        pltpu.async_copy(tok_hbm.at[idx_v.at[s // 8, 0, s % 8]], rows[b], gsems[b])

    def drain_gather(b):
        pltpu.make_async_copy(tok_hbm.at[pl.ds(0, BB)], rows[b], gsems[b]).wait()

    def fire_out(s, b):
        pltpu.async_copy(tbs[b], out_hbm.at[s, :, pl.ds(wid, 1)], osems[b])

    def drain_out(b):
        pltpu.make_async_copy(tbs[b], out_hbm.at[0, :, pl.ds(0, 1)], osems[b]).wait()

    def compute(s, b):
        rb, tb = rows[b], tbs[b]
        s_splat = jnp.full((16,), s, jnp.int32)
        cvecs = [iota + 16 * j for j in range(BB // 16)]

        @plsc.parallel_loop(0, 16, 1)
        def _(t):
            sl = pl.ds(0, 16)
            m = skew_v[0, t, sl]
            kb = skew_v[1, t, sl]
            wb = skew_v[2, t, sl]
            for u in range(DIM // 16):
                dvec = 16 * u + m
                pg = plsc.load_gather(pos_v, [s_splat, dvec])
                kvec = 2 * u + kb
                for j in range(BB // 16):
                    g = plsc.load_gather(rb, [cvecs[j], dvec])
                    plsc.store_scatter(tb, [kvec, zero16, 16 * j + wb],
                                       g * 8.0 + pg)

    fire_gather(0, 0)
    fire_gather(1, 1)

    def step(s, b):
        @pl.when(s >= 2)
        def _():
            drain_out(b)

        drain_gather(b)
        compute(s, b)
        fire_out(s, b)

        @pl.when(s + 2 < seq)
        def _():
            fire_gather(s + 2, b)

    def pair(t, _):
        step(2 * t, 0)
        step(2 * t + 1, 1)
        return _

    lax.fori_loop(0, seq // 2, pair, 0)
    drain_out(0)
    drain_out(1)


def kernel(inputs, token_table, position_table):
    batch, seq = inputs.shape
    vocab, dim = token_table.shape
    idx4 = (
        inputs.astype(jnp.int32)
        .reshape(batch // BB, BB, seq // 8, 8)
        .transpose(2, 0, 3, 1)
    )

    mesh = plsc.VectorSubcoreMesh(
        core_axis_name="c", subcore_axis_name="s", num_cores=NC, num_subcores=NS
    )
    call = pl.kernel(
        _sc_body,
        out_type=jax.ShapeDtypeStruct((seq, dim // 8, batch // BB, 8 * BB), jnp.float32),
        name="emb_kernel",
        mesh=mesh,
        scratch_types=[
            pltpu.VMEM((seq // 8, 1, 8, BB), jnp.int32),
            pltpu.VMEM((BB, dim), jnp.float32),
            pltpu.VMEM((BB, dim), jnp.float32),
            pltpu.VMEM((dim // 8, 1, 8 * BB), jnp.float32),
            pltpu.VMEM((dim // 8, 1, 8 * BB), jnp.float32),
            pltpu.VMEM((seq, dim), jnp.float32),
            pltpu.VMEM((3, 16, 16), jnp.int32),
            pltpu.SemaphoreType.DMA,
            pltpu.SemaphoreType.DMA,
            pltpu.SemaphoreType.DMA,
            pltpu.SemaphoreType.DMA,
        ],
        compiler_params=pltpu.CompilerParams(
            use_tc_tiling_on_sc=False, needs_layout_passes=False
        ),
    )
    a = call(idx4, token_table, position_table)
    return (
        a.reshape(seq, dim // 8, batch // BB, 8, BB)
        .transpose(2, 4, 0, 1, 3)
        .reshape(batch, seq, dim)
    )
